# baseline (speedup 1.0000x reference)
import functools

import jax
import jax.numpy as jnp
from jax import lax
from jax.experimental import pallas as pl
from jax.experimental.pallas import tpu as pltpu
from jax.experimental.pallas import tpu_sc as plsc

N_CODES = 8192
N_POINTS = 8192
DIM = 64
BLK_Z = 1024
BLK_C = 1024
N_CB_BLOCKS = N_CODES // BLK_C


def _dist_argmin_body(z_ref, cbt_ref, zz_ref, ee_ref, out_ref, best_val, best_idx):
    j = pl.program_id(1)
    dot = lax.dot_general(
        z_ref[...].astype(jnp.bfloat16), cbt_ref[...].astype(jnp.bfloat16),
        dimension_numbers=(((1,), (0,)), ((), ())),
        preferred_element_type=jnp.float32,
    )
    d = (zz_ref[...] - 2.0 * dot) + ee_ref[...]
    m = jnp.min(d, axis=1, keepdims=True)
    lane = lax.broadcasted_iota(jnp.int32, (BLK_Z, BLK_C), 1) + j * BLK_C
    idx_local = jnp.min(
        jnp.where(d == m, lane, jnp.int32(2**30)), axis=1, keepdims=True
    )

    @pl.when(j == 0)
    def _():
        best_val[...] = m
        best_idx[...] = idx_local

    @pl.when(j > 0)
    def _():
        better = m < best_val[...]
        best_idx[...] = jnp.where(better, idx_local, best_idx[...])
        best_val[...] = jnp.minimum(m, best_val[...])

    @pl.when(j == N_CB_BLOCKS // 2 - 1)
    def _():
        best_val[...] = best_val[...].astype(jnp.bfloat16).astype(jnp.float32)

    @pl.when(j == N_CB_BLOCKS - 1)
    def _():
        out_ref[...] = best_idx[...]


def _tc_argmin(z_flat, cbt, zz, ee):
    return pl.pallas_call(
        _dist_argmin_body,
        grid=(N_POINTS // BLK_Z, N_CB_BLOCKS),
        in_specs=[
            pl.BlockSpec((BLK_Z, DIM), lambda i, j: (i, 0)),
            pl.BlockSpec((DIM, BLK_C), lambda i, j: (0, j)),
            pl.BlockSpec((BLK_Z, 1), lambda i, j: (i, 0)),
            pl.BlockSpec((1, BLK_C), lambda i, j: (0, j)),
        ],
        out_specs=pl.BlockSpec((BLK_Z, 1), lambda i, j: (i, 0)),
        out_shape=jax.ShapeDtypeStruct((N_POINTS, 1), jnp.int32),
        scratch_shapes=[
            pltpu.VMEM((BLK_Z, 1), jnp.float32),
            pltpu.VMEM((BLK_Z, 1), jnp.int32),
        ],
        compiler_params=pltpu.CompilerParams(
            dimension_semantics=("parallel", "arbitrary"),
        ),
    )(z_flat, cbt, zz, ee)


_NC, _NS = 2, 16
_NW = _NC * _NS
_ROWS_PER_W = N_POINTS // _NW
DIM_PAD = 128


@functools.cache
def _make_sc_gather():
    @functools.partial(
        pl.kernel,
        mesh=plsc.VectorSubcoreMesh(core_axis_name="c", subcore_axis_name="s"),
        out_type=jax.ShapeDtypeStruct((N_POINTS, DIM_PAD), jnp.float32),
        scratch_types=[
            pltpu.VMEM((_ROWS_PER_W,), jnp.int32),
            pltpu.VMEM((_ROWS_PER_W, DIM_PAD), jnp.float32),
            pltpu.SemaphoreType.DMA,
        ],
    )
    def _sc_gather(table_hbm, idx_hbm, out_hbm, idx_v, rows_v, sem):
        wid = lax.axis_index("s") * _NC + lax.axis_index("c")
        base = wid * _ROWS_PER_W
        pltpu.sync_copy(idx_hbm.at[pl.ds(base, _ROWS_PER_W)], idx_v)
        pltpu.async_copy(table_hbm.at[idx_v], rows_v, sem).wait()
        pltpu.sync_copy(rows_v, out_hbm.at[pl.ds(base, _ROWS_PER_W)])

    return _sc_gather


def kernel(z_e, codebook):
    B, C, H, W = z_e.shape
    z_flat = jnp.transpose(z_e, (0, 2, 3, 1)).reshape(-1, C)
    zz = jnp.sum(z_flat * z_flat, axis=1, keepdims=True)
    ee = jnp.sum(codebook * codebook, axis=1)[None, :]
    indices = _tc_argmin(z_flat, codebook.T, zz, ee).reshape(-1)
    cb_pad = jnp.pad(codebook, ((0, 0), (0, DIM_PAD - DIM)))
    z_q_flat = _make_sc_gather()(cb_pad, indices)[:, :DIM]
    z_q = jnp.transpose(z_q_flat.reshape(B, H, W, C), (0, 3, 1, 2))
    z_q_st = z_e + lax.stop_gradient(z_q - z_e)
    indices_out = indices.reshape(B, H * W)
    return (z_q_st, indices_out, z_q)
